# baseline (speedup 1.0000x reference)
import functools

import numpy as np

import jax
import jax.numpy as jnp
from jax.experimental import pallas as pl
from jax.experimental.pallas import tpu as pltpu


def _interp_matrix(out_size: int, in_size: int) -> np.ndarray:
    scale = np.float32(in_size / out_size)
    o = np.arange(out_size, dtype=np.float32)
    src = np.clip((o + np.float32(0.5)) * scale - np.float32(0.5),
                  np.float32(0.0), np.float32(in_size - 1))
    lo = np.floor(src).astype(np.int32)
    hi = np.minimum(lo + 1, in_size - 1)
    frac = (src - lo.astype(np.float32)).astype(np.float32)
    rows = np.arange(out_size)
    m = np.zeros((out_size, in_size), dtype=np.float32)
    np.add.at(m, (rows, lo), np.float32(1.0) - frac)
    np.add.at(m, (rows, hi), frac)
    return m


def _resize_body(a_ref, bt_ref, x_ref, o_ref, tmp_ref):
    bp, h, w = x_ref.shape
    s = o_ref.shape[1]
    xb = x_ref[...].astype(jnp.bfloat16).reshape(bp * h, w)
    tmp_ref[...] = (
        jnp.dot(xb, bt_ref[...], preferred_element_type=jnp.float32)
        .astype(jnp.bfloat16)
        .reshape(bp, h, -1)
    )
    a = a_ref[...]
    for b in range(bp):
        o_ref[b] = jnp.dot(
            a, tmp_ref[b], preferred_element_type=jnp.float32
        )[:, :s]


@functools.partial(jax.jit, static_argnums=(1, 2))
def _resize_planes(x_planes: jnp.ndarray, s: int, bp: int) -> jnp.ndarray:
    nc, h, w = x_planes.shape
    np_lanes = ((s + 255) // 256) * 256
    a = jnp.asarray(_interp_matrix(s, h), dtype=jnp.bfloat16)
    bt_np = np.zeros((w, np_lanes), dtype=np.float32)
    bt_np[:, :s] = _interp_matrix(s, w).T
    bt = jnp.asarray(bt_np, dtype=jnp.bfloat16)
    return pl.pallas_call(
        _resize_body,
        out_shape=jax.ShapeDtypeStruct((nc, s, s), x_planes.dtype),
        grid=(nc // bp,),
        in_specs=[
            pl.BlockSpec((s, h), lambda i: (0, 0)),
            pl.BlockSpec((w, np_lanes), lambda i: (0, 0)),
            pl.BlockSpec((bp, h, w), lambda i: (i, 0, 0)),
        ],
        out_specs=pl.BlockSpec((bp, s, s), lambda i: (i, 0, 0)),
        scratch_shapes=[pltpu.VMEM((bp, h, np_lanes), jnp.bfloat16)],
        compiler_params=pltpu.CompilerParams(
            dimension_semantics=("parallel",),
            vmem_limit_bytes=64 << 20,
        ),
    )(a, bt, x_planes)


def kernel(x):
    n, c, h, w = x.shape
    s = 224
    nc = n * c
    bp = 48 if nc % 48 == 0 else (8 if nc % 8 == 0 else 1)
    out = _resize_planes(x.reshape(nc, h, w), s, bp)
    return out.reshape(n, c, s, s)


# speedup vs baseline: 1.0054x; 1.0054x over previous
import functools

import numpy as np

import jax
import jax.numpy as jnp
from jax.experimental import pallas as pl
from jax.experimental.pallas import tpu as pltpu


def _interp_matrix(out_size: int, in_size: int) -> np.ndarray:
    scale = np.float32(in_size / out_size)
    o = np.arange(out_size, dtype=np.float32)
    src = np.clip((o + np.float32(0.5)) * scale - np.float32(0.5),
                  np.float32(0.0), np.float32(in_size - 1))
    lo = np.floor(src).astype(np.int32)
    hi = np.minimum(lo + 1, in_size - 1)
    frac = (src - lo.astype(np.float32)).astype(np.float32)
    rows = np.arange(out_size)
    m = np.zeros((out_size, in_size), dtype=np.float32)
    np.add.at(m, (rows, lo), np.float32(1.0) - frac)
    np.add.at(m, (rows, hi), frac)
    return m


def _resize_body(a_ref, bt_ref, x_ref, o_ref, tmp_ref):
    bp, h, w = x_ref.shape
    s = o_ref.shape[1]
    xb = x_ref[...].astype(jnp.bfloat16).reshape(bp * h, w)
    tmp_ref[...] = (
        jnp.dot(xb, bt_ref[...], preferred_element_type=jnp.float32)
        .astype(jnp.bfloat16)
        .reshape(bp, h, -1)
    )
    a = a_ref[...]
    for b in range(bp):
        o_ref[b] = jnp.dot(
            a, tmp_ref[b], preferred_element_type=jnp.float32
        )[:, :s]


@functools.partial(jax.jit, static_argnums=(1, 2))
def _resize_planes(x_planes: jnp.ndarray, s: int, bp: int) -> jnp.ndarray:
    nc, h, w = x_planes.shape
    np_lanes = ((s + 255) // 256) * 256
    a = jnp.asarray(_interp_matrix(s, h), dtype=jnp.bfloat16)
    bt_np = np.zeros((w, np_lanes), dtype=np.float32)
    bt_np[:, :s] = _interp_matrix(s, w).T
    bt = jnp.asarray(bt_np, dtype=jnp.bfloat16)
    return pl.pallas_call(
        _resize_body,
        out_shape=jax.ShapeDtypeStruct((nc, s, s), x_planes.dtype),
        grid=(nc // bp,),
        in_specs=[
            pl.BlockSpec((s, h), lambda i: (0, 0)),
            pl.BlockSpec((w, np_lanes), lambda i: (0, 0)),
            pl.BlockSpec((bp, h, w), lambda i: (i, 0, 0)),
        ],
        out_specs=pl.BlockSpec((bp, s, s), lambda i: (i, 0, 0)),
        scratch_shapes=[pltpu.VMEM((bp, h, np_lanes), jnp.bfloat16)],
        compiler_params=pltpu.CompilerParams(
            dimension_semantics=("parallel",),
            vmem_limit_bytes=64 << 20,
        ),
    )(a, bt, x_planes)


def kernel(x):
    n, c, h, w = x.shape
    s = 224
    nc = n * c
    bp = 32 if nc % 32 == 0 else (8 if nc % 8 == 0 else 1)
    out = _resize_planes(x.reshape(nc, h, w), s, bp)
    return out.reshape(n, c, s, s)


# speedup vs baseline: 1.0082x; 1.0029x over previous
import functools

import numpy as np

import jax
import jax.numpy as jnp
from jax.experimental import pallas as pl
from jax.experimental.pallas import tpu as pltpu


def _interp_matrix(out_size: int, in_size: int) -> np.ndarray:
    scale = np.float32(in_size / out_size)
    o = np.arange(out_size, dtype=np.float32)
    src = np.clip((o + np.float32(0.5)) * scale - np.float32(0.5),
                  np.float32(0.0), np.float32(in_size - 1))
    lo = np.floor(src).astype(np.int32)
    hi = np.minimum(lo + 1, in_size - 1)
    frac = (src - lo.astype(np.float32)).astype(np.float32)
    rows = np.arange(out_size)
    m = np.zeros((out_size, in_size), dtype=np.float32)
    np.add.at(m, (rows, lo), np.float32(1.0) - frac)
    np.add.at(m, (rows, hi), frac)
    return m


def _resize_body(a_ref, bt_ref, x_ref, o_ref, tmp_ref):
    bp, h, w = x_ref.shape
    s = o_ref.shape[1]
    xb = x_ref[...].astype(jnp.bfloat16).reshape(bp * h, w)
    tmp_ref[...] = (
        jnp.dot(xb, bt_ref[...], preferred_element_type=jnp.float32)
        .astype(jnp.bfloat16)
        .reshape(bp, h, -1)
    )
    a = a_ref[...]
    for b in range(bp):
        o_ref[b] = jnp.dot(
            a, tmp_ref[b], preferred_element_type=jnp.float32
        )[:, :s]


@functools.partial(jax.jit, static_argnums=(1, 2))
def _resize_planes(x_planes: jnp.ndarray, s: int, bp: int) -> jnp.ndarray:
    nc, h, w = x_planes.shape
    np_lanes = ((s + 255) // 256) * 256
    a = jnp.asarray(_interp_matrix(s, h), dtype=jnp.bfloat16)
    bt_np = np.zeros((w, np_lanes), dtype=np.float32)
    bt_np[:, :s] = _interp_matrix(s, w).T
    bt = jnp.asarray(bt_np, dtype=jnp.bfloat16)
    return pl.pallas_call(
        _resize_body,
        out_shape=jax.ShapeDtypeStruct((nc, s, s), x_planes.dtype),
        grid=(nc // bp,),
        in_specs=[
            pl.BlockSpec((s, h), lambda i: (0, 0)),
            pl.BlockSpec((w, np_lanes), lambda i: (0, 0)),
            pl.BlockSpec((bp, h, w), lambda i: (i, 0, 0)),
        ],
        out_specs=pl.BlockSpec((bp, s, s), lambda i: (i, 0, 0)),
        scratch_shapes=[pltpu.VMEM((bp, h, np_lanes), jnp.bfloat16)],
        compiler_params=pltpu.CompilerParams(
            dimension_semantics=("parallel",),
            vmem_limit_bytes=64 << 20,
        ),
    )(a, bt, x_planes)


def kernel(x):
    n, c, h, w = x.shape
    s = 224
    nc = n * c
    bp = 24 if nc % 24 == 0 else (8 if nc % 8 == 0 else 1)
    out = _resize_planes(x.reshape(nc, h, w), s, bp)
    return out.reshape(n, c, s, s)
